# baseline (speedup 1.0000x reference)
import jax
import jax.numpy as jnp
from jax import lax
from jax.experimental import pallas as pl
from jax.experimental.pallas import tpu as pltpu
from jax.experimental.pallas import tpu_sc as plsc

N_NODES = 10000
N_EDGES = 320000
D = 128

NC = 2
NS = 16
L = 16
NW = NC * NS
CHUNK = 128
NCH = 80
E_PAD = NW * NCH * CHUNK
RPT = 624
TAIL = N_NODES - NS * RPT

_SELU_SCALE = 1.0507009873554805
_SELU_ALPHA = 1.6732632423543772


HALF = NCH // 2


def _sc_agg_body(feat_hbm, rows_hbm, cols_hbm, vals_hbm, zeros_hbm, out_hbm,
                 cols_v, rows_v, vals_v, g0, g1, spmem_agg, sem0, sem1):
    cid = lax.axis_index("c")
    sid = lax.axis_index("s")
    wid = cid * NS + sid

    zoff = pl.multiple_of(sid * RPT, 8)
    pltpu.sync_copy(zeros_hbm.at[pl.ds(zoff, RPT)],
                    spmem_agg.at[pl.ds(zoff, RPT)])
    @pl.when(sid == NS - 1)
    def _():
        pltpu.sync_copy(zeros_hbm.at[pl.ds(NS * RPT, TAIL)],
                        spmem_agg.at[pl.ds(NS * RPT, TAIL)])
    plsc.subcore_barrier()

    bufs = (g0, g1)
    sems = (sem0, sem1)

    def start_gather(g, b):
        pltpu.async_copy(feat_hbm.at[cols_v.at[g]], bufs[b], sems[b])

    def wait_gather(b):
        pltpu.make_async_copy(feat_hbm.at[pl.ds(0, CHUNK)], bufs[b],
                              sems[b]).wait()

    def process(g, b):
        buf = bufs[b]
        wait_gather(b)

        def grp_body(k, c2):
            vgrp = vals_v[g, pl.ds(k * L, L)]
            for t in range(L):
                v = vgrp[t]
                e = k * L + t
                for j in range(D // L):
                    sl = pl.ds(j * L, L)
                    buf[e, sl] = buf[e, sl] * v
            return c2
        lax.fori_loop(0, CHUNK // L, grp_body, 0, unroll=False)

        pltpu.sync_copy(buf, spmem_agg.at[rows_v.at[g]], add=True)

    for h in range(2):
        cbase = pl.multiple_of(wid * NCH + h * HALF, 8)
        pltpu.sync_copy(cols_hbm.at[pl.ds(cbase, HALF)], cols_v)
        pltpu.sync_copy(rows_hbm.at[pl.ds(cbase, HALF)], rows_v)
        pltpu.sync_copy(vals_hbm.at[pl.ds(cbase, HALF)], vals_v)

        start_gather(0, 0)

        def pair_body(i, c):
            ga = i * 2
            start_gather(ga + 1, 1)
            process(ga, 0)
            start_gather(ga + 2, 0)
            process(ga + 1, 1)
            return c
        lax.fori_loop(0, HALF // 2 - 1, pair_body, 0, unroll=False)

        start_gather(HALF - 1, 1)
        process(HALF - 2, 0)
        process(HALF - 1, 1)

    plsc.subcore_barrier()

    woff = pl.multiple_of(sid * RPT, 8)
    pltpu.sync_copy(spmem_agg.at[pl.ds(woff, RPT)],
                    out_hbm.at[cid, pl.ds(woff, RPT)])
    @pl.when(sid == NS - 1)
    def _():
        pltpu.sync_copy(spmem_agg.at[pl.ds(NS * RPT, TAIL)],
                        out_hbm.at[cid, pl.ds(NS * RPT, TAIL)])


def _sc_aggregate(features, rows2, cols2, vals2, zeros):
    mesh = plsc.VectorSubcoreMesh(core_axis_name="c", subcore_axis_name="s")
    f = pl.kernel(
        _sc_agg_body,
        out_type=jax.ShapeDtypeStruct((NC, N_NODES, D), jnp.float32),
        mesh=mesh,
        scratch_types=[
            pltpu.VMEM((HALF, CHUNK), jnp.int32),
            pltpu.VMEM((HALF, CHUNK), jnp.int32),
            pltpu.VMEM((HALF, CHUNK), jnp.float32),
            pltpu.VMEM((CHUNK, D), jnp.float32),
            pltpu.VMEM((CHUNK, D), jnp.float32),
            pltpu.VMEM_SHARED((N_NODES, D), jnp.float32),
            pltpu.SemaphoreType.DMA,
            pltpu.SemaphoreType.DMA,
        ],
    )
    return f(features, rows2, cols2, vals2, zeros)


def _tc_body(f_ref, p_ref, k_ref, b_ref, sw_ref, o_ref):
    h = jnp.dot(f_ref[...], k_ref[...], preferred_element_type=jnp.float32,
                precision=lax.Precision.HIGHEST)
    agg = jnp.dot(p_ref[0] + p_ref[1], k_ref[...],
                  preferred_element_type=jnp.float32,
                  precision=lax.Precision.HIGHEST)
    y = h * sw_ref[...] + agg + b_ref[...]
    o_ref[...] = jnp.where(
        y > 0.0,
        _SELU_SCALE * y,
        (_SELU_SCALE * _SELU_ALPHA) * (jnp.exp(jnp.minimum(y, 0.0)) - 1.0),
    )


def _tc_finish(features, partials, k, bias2, sw2):
    BM = 2000
    return pl.pallas_call(
        _tc_body,
        grid=(N_NODES // BM,),
        in_specs=[
            pl.BlockSpec((BM, D), lambda i: (i, 0)),
            pl.BlockSpec((NC, BM, D), lambda i: (0, i, 0)),
            pl.BlockSpec((D, D), lambda i: (0, 0)),
            pl.BlockSpec((1, D), lambda i: (0, 0)),
            pl.BlockSpec((1, D), lambda i: (0, 0)),
        ],
        out_specs=pl.BlockSpec((BM, D), lambda i: (i, 0)),
        out_shape=jax.ShapeDtypeStruct((N_NODES, D), jnp.float32),
    )(features, partials, k, bias2, sw2)


def kernel(features, adj_indices, adj_values, kernel, bias, skip_weight):
    pad = E_PAD - N_EDGES
    idx2 = jnp.pad(adj_indices, ((0, 0), (0, pad)))
    rows2 = idx2[0].reshape(NW * NCH, CHUNK)
    cols2 = idx2[1].reshape(NW * NCH, CHUNK)
    vals2 = jnp.pad(adj_values, (0, pad)).reshape(NW * NCH, CHUNK)
    zeros = jnp.zeros((N_NODES, D), jnp.float32)
    partials = _sc_aggregate(features, rows2, cols2, vals2, zeros)
    return _tc_finish(features, partials, kernel,
                      bias.reshape(1, D), skip_weight.reshape(1, D))
